# baseline (speedup 1.0000x reference)
import jax
import jax.numpy as jnp
from jax import lax
from jax.experimental import pallas as pl
from jax.experimental.pallas import tpu as pltpu
from jax.experimental.pallas import tpu_sc as plsc

NC = 2
NS = 16
NW = NC * NS
CPERW = 128
CPCHUNK = 2
NBUF = 4


def _body(x_hbm, table_hbm, out_hbm, idx_v, table_v, table_sh, bufs,
          gsems, wsems):
    sid = lax.axis_index("s")
    wid = sid * NC + lax.axis_index("c")
    nrowchunks = x_hbm.shape[1]
    nchunks = nrowchunks // CPCHUNK
    ngroups = nchunks // NBUF

    @pl.when(sid == 0)
    def _():
        pltpu.sync_copy(table_hbm, table_v)
        pltpu.sync_copy(table_v, table_sh)

    pltpu.sync_copy(x_hbm.at[wid], idx_v)
    plsc.subcore_barrier()

    def gather(c, b):
        for k in range(CPCHUNK):
            pltpu.make_async_copy(
                table_sh.at[idx_v.at[CPCHUNK * c + k]],
                bufs[b].at[k], gsems[b]).start()

    def wait_gather(b):
        for k in range(CPCHUNK):
            pltpu.make_async_copy(
                table_sh.at[idx_v.at[0]], bufs[b].at[k], gsems[b]).wait()

    def write(c, b):
        pltpu.make_async_copy(
            bufs[b], out_hbm.at[pl.ds(wid * CPERW + CPCHUNK * c, CPCHUNK)],
            wsems[b]).start()

    def wait_write(b):
        pltpu.make_async_copy(
            bufs[b], out_hbm.at[pl.ds(0, CPCHUNK)], wsems[b]).wait()

    for b in range(NBUF):
        gather(b, b)

    def step(g, carry):
        c0 = NBUF * g
        for b in range(NBUF):
            wait_gather(b)
            write(c0 + b, b)
        for b in range(NBUF):
            wait_write(b)
            gather(c0 + NBUF + b, b)
        return carry

    lax.fori_loop(0, ngroups - 1, step, 0)

    c0 = (ngroups - 1) * NBUF
    for b in range(NBUF):
        wait_gather(b)
        write(c0 + b, b)
    for b in range(NBUF):
        wait_write(b)


def kernel(x, table):
    B0, B1 = x.shape
    V, D = table.shape
    x3 = x.reshape(NW, CPERW, B1)

    fn = pl.kernel(
        _body,
        out_type=jax.ShapeDtypeStruct((B0, B1, D), jnp.float32),
        mesh=plsc.VectorSubcoreMesh(core_axis_name="c", subcore_axis_name="s"),
        compiler_params=pltpu.CompilerParams(use_tc_tiling_on_sc=False),
        scratch_types=[
            pltpu.VMEM((CPERW, B1), jnp.int32),
            pltpu.VMEM((V, D), jnp.float32),
            pltpu.VMEM_SHARED((V, D), jnp.float32),
            [pltpu.VMEM((CPCHUNK, B1, D), jnp.float32) for _ in range(NBUF)],
            [pltpu.SemaphoreType.DMA for _ in range(NBUF)],
            [pltpu.SemaphoreType.DMA for _ in range(NBUF)],
        ],
    )
    return fn(x3, table)


# speedup vs baseline: 1.1010x; 1.1010x over previous
import jax
import jax.numpy as jnp
import numpy as np
from jax import lax
from jax.experimental import pallas as pl
from jax.experimental.pallas import tpu as pltpu
from jax.experimental.pallas import tpu_sc as plsc

NC = 2
NS = 16
NW = NC * NS
NBUF = 4
LANE = 128
SUB = 8


def _body(idx2_hbm, table2_hbm, pat_hbm, out_hbm, idx_v, pat_v, table_v,
          table_sh, bufs, gsems, wsems):
    sid = lax.axis_index("s")
    wid = sid * NC + lax.axis_index("c")
    cperw = idx2_hbm.shape[1]
    nseg = idx2_hbm.shape[2]

    @pl.when(sid == 0)
    def _():
        pltpu.sync_copy(table2_hbm, table_v)
        pltpu.sync_copy(table_v, table_sh)

    pltpu.sync_copy(idx2_hbm.at[wid], idx_v)
    pltpu.sync_copy(pat_hbm, pat_v)
    plsc.subcore_barrier()

    def gather(c, b):
        pltpu.make_async_copy(
            table_sh.at[idx_v.at[c]], bufs[b], gsems[b]).start()

    def wait_gather(b):
        pltpu.make_async_copy(
            table_sh.at[idx_v.at[0]], bufs[b], gsems[b]).wait()

    def scatter(c, b):
        pltpu.make_async_copy(
            bufs[b], out_hbm.at[wid * cperw + c].at[pat_v.at[0]],
            wsems[b]).start()

    def wait_scatter(b):
        pltpu.make_async_copy(
            bufs[b], out_hbm.at[0].at[pat_v.at[0]], wsems[b]).wait()

    for b in range(NBUF):
        gather(b, b)

    def step(g, carry):
        c0 = NBUF * g
        for b in range(NBUF):
            wait_gather(b)
            scatter(c0 + b, b)
        for b in range(NBUF):
            wait_scatter(b)
            gather(c0 + NBUF + b, b)
        return carry

    lax.fori_loop(0, cperw // NBUF - 1, step, 0)

    c0 = cperw - NBUF
    for b in range(NBUF):
        wait_gather(b)
        scatter(c0 + b, b)
    for b in range(NBUF):
        wait_scatter(b)


def kernel(x, table):
    B0, B1 = x.shape
    V, D = table.shape
    cperw = B0 // NW
    jt = (B1 + SUB - 1) // SUB
    dt = (D + LANE - 1) // LANE
    nseg = B1 * dt

    table2 = jnp.pad(table, ((0, 0), (0, dt * LANE - D))).reshape(
        V * dt, LANE)

    x3 = x.reshape(NW, cperw, B1)
    idx2 = jnp.stack([2 * x3, 2 * x3 + 1], axis=-1).reshape(NW, cperw, nseg)

    j = np.arange(B1)
    base = (j // SUB) * (dt * SUB) + (j % SUB)
    pat = np.stack([base, base + SUB], axis=-1).reshape(1, nseg)
    pat = jnp.asarray(pat, dtype=jnp.int32)

    fn = pl.kernel(
        _body,
        out_type=jax.ShapeDtypeStruct((B0, jt * dt * SUB, LANE), jnp.float32),
        mesh=plsc.VectorSubcoreMesh(core_axis_name="c", subcore_axis_name="s"),
        compiler_params=pltpu.CompilerParams(use_tc_tiling_on_sc=False),
        scratch_types=[
            pltpu.VMEM((cperw, nseg), jnp.int32),
            pltpu.VMEM((1, nseg), jnp.int32),
            pltpu.VMEM((V * dt, LANE), jnp.float32),
            pltpu.VMEM_SHARED((V * dt, LANE), jnp.float32),
            [pltpu.VMEM((nseg, LANE), jnp.float32) for _ in range(NBUF)],
            [pltpu.SemaphoreType.DMA for _ in range(NBUF)],
            [pltpu.SemaphoreType.DMA for _ in range(NBUF)],
        ],
    )
    t5 = fn(idx2, table2, pat)
    out = t5.reshape(B0, jt, dt, SUB, LANE).transpose(0, 1, 3, 2, 4)
    out = out.reshape(B0, jt * SUB, dt * LANE)[:, :B1, :D]
    return out
